# baseline (speedup 1.0000x reference)
import functools

import jax
import jax.numpy as jnp
from jax import lax
from jax.experimental import pallas as pl
from jax.experimental.pallas import tpu as pltpu
from jax.experimental.pallas import tpu_sc as plsc

N_NODES = 10000
N_EDGES = 320000
NC, NS = 2, 16
NW = NC * NS
N_PAD = 10240
JUNK = N_NODES
SLOTS = 10240
CHUNK = 2048
NCH = SLOTS // CHUNK
EPT = N_EDGES // NW
ROWS_T = N_PAD // NS

_mesh = plsc.VectorSubcoreMesh(
    core_axis_name="c", subcore_axis_name="s", num_cores=NC, num_subcores=NS)


def _make_deg_kernel():
    @functools.partial(
        pl.kernel,
        out_type=jax.ShapeDtypeStruct((NC * N_PAD, 16), jnp.float32),
        mesh=_mesh,
        scratch_types=[
            pltpu.VMEM((NCH, CHUNK), jnp.int32),
            pltpu.VMEM((CHUNK, 16), jnp.float32),
            pltpu.VMEM_SHARED((N_PAD, 16), jnp.float32),
        ],
        compiler_params=pltpu.CompilerParams(use_tc_tiling_on_sc=False),
    )
    def k(dst_hbm, zeros_hbm, ones_hbm, z_hbm, dst_v, rows_v, z_sh):
        cid = lax.axis_index("c")
        sid = lax.axis_index("s")
        g = cid * NS + sid
        pltpu.sync_copy(dst_hbm.at[g], dst_v)
        pltpu.sync_copy(zeros_hbm, rows_v)
        base = sid * ROWS_T
        for kk in range(ROWS_T // 128):
            pltpu.sync_copy(rows_v.at[pl.ds(0, 128)],
                            z_sh.at[pl.ds(base + kk * 128, 128)])
        pltpu.sync_copy(ones_hbm, rows_v)
        plsc.subcore_barrier()

        def chunk_body(j, carry):
            pltpu.sync_copy(rows_v, z_sh.at[dst_v.at[j]], add=True)
            return carry

        lax.fori_loop(0, NCH, chunk_body, 0)
        plsc.subcore_barrier()
        pltpu.sync_copy(z_sh.at[pl.ds(base, ROWS_T)],
                        z_hbm.at[pl.ds(cid * N_PAD + base, ROWS_T)])

    return k


def _make_scatter_kernel(d, chunk, slots, per_core_edges):
    nch = slots // chunk

    @functools.partial(
        pl.kernel,
        out_type=jax.ShapeDtypeStruct((NC * N_PAD, d), jnp.float32),
        mesh=_mesh,
        scratch_types=[
            pltpu.VMEM((nch, chunk), jnp.int32),
            pltpu.VMEM((nch, chunk), jnp.int32),
            pltpu.VMEM((2, chunk, d), jnp.float32),
            pltpu.VMEM_SHARED((N_PAD, d), jnp.float32),
            pltpu.VMEM_SHARED((N_PAD, d), jnp.float32),
            pltpu.SemaphoreType.DMA,
        ],
        compiler_params=pltpu.CompilerParams(use_tc_tiling_on_sc=False),
    )
    def k(src_hbm, dst_hbm, y_hbm, zeros_hbm, z_hbm,
          src_v, dst_v, rows_v, z_sh, y_sh, sem):
        cid = lax.axis_index("c")
        sid = lax.axis_index("s")
        g = sid if per_core_edges else cid * NS + sid
        pltpu.sync_copy(src_hbm.at[g], src_v)
        pltpu.sync_copy(dst_hbm.at[g], dst_v)
        base = sid * ROWS_T
        yoff = cid * N_PAD + base if per_core_edges else base
        pltpu.sync_copy(y_hbm.at[pl.ds(yoff, ROWS_T)],
                        y_sh.at[pl.ds(base, ROWS_T)])
        pltpu.sync_copy(zeros_hbm, rows_v.at[0])
        for kk in range(ROWS_T // 128):
            pltpu.sync_copy(rows_v.at[0].at[pl.ds(0, 128)],
                            z_sh.at[pl.ds(base + kk * 128, 128)])
        plsc.subcore_barrier()

        pltpu.async_copy(y_sh.at[src_v.at[0]], rows_v.at[0], sem)
        for j in range(nch):
            b = j % 2
            pltpu.make_async_copy(
                y_sh.at[src_v.at[j]], rows_v.at[b], sem).wait()
            if j + 1 < nch:
                pltpu.async_copy(
                    y_sh.at[src_v.at[j + 1]], rows_v.at[1 - b], sem)
            pltpu.sync_copy(rows_v.at[b], z_sh.at[dst_v.at[j]], add=True)
        plsc.subcore_barrier()
        pltpu.sync_copy(z_sh.at[pl.ds(base, ROWS_T)],
                        z_hbm.at[pl.ds(cid * N_PAD + base, ROWS_T)])

    return k


_CH64 = 512
_SL64 = 20480
_CH16 = 1024
_SL16 = SLOTS

_deg_kernel = _make_deg_kernel()
_scatter64 = _make_scatter_kernel(32, _CH64, _SL64, per_core_edges=True)
_scatter16 = _make_scatter_kernel(16, _CH16, _SL16, per_core_edges=False)

def _dis_of(dp_ref):
    deg = dp_ref[0, :, 0:1] + dp_ref[1, :, 0:1] + 1.0
    return lax.rsqrt(deg)


def _y1_body(x_ref, w_ref, dp_ref, y_ref):
    dis = _dis_of(dp_ref)
    y = jnp.dot(x_ref[...] * dis, w_ref[...],
                preferred_element_type=jnp.float32)
    y_ref[0] = y[:, 0:32]
    y_ref[1] = y[:, 32:64]


def _tc_y1(x_pad, W1, degp):
    return pl.pallas_call(
        _y1_body,
        out_shape=jax.ShapeDtypeStruct((2, N_PAD, 32), jnp.float32),
    )(x_pad, W1, degp)


def _y2_body(z_ref, y1_ref, dp_ref, w_ref, b_ref, y2_ref):
    dis = _dis_of(dp_ref)
    zc = jnp.concatenate([z_ref[0], z_ref[1]], axis=-1)
    yc = jnp.concatenate([y1_ref[0], y1_ref[1]], axis=-1)
    s = dis * (zc + yc) + b_ref[...]
    h = jnp.maximum(s, 0.0) * dis
    y2_ref[...] = jnp.dot(h, w_ref[...], preferred_element_type=jnp.float32)


def _tc_y2(z1, y1, degp, W2, b1):
    return pl.pallas_call(
        _y2_body,
        out_shape=jax.ShapeDtypeStruct((N_PAD, 16), jnp.float32),
    )(z1, y1, degp, W2, b1)


def _out_body(z_ref, y2_ref, dp_ref, b_ref, o_ref):
    dis = _dis_of(dp_ref)
    s = dis * (z_ref[0] + z_ref[1] + y2_ref[...]) + b_ref[...]
    m = jnp.max(s, axis=1, keepdims=True)
    e = jnp.exp(s - m)
    o_ref[...] = s - (m + jnp.log(jnp.sum(e, axis=1, keepdims=True)))


def _tc_out(z2, y2, degp, b2):
    return pl.pallas_call(
        _out_body,
        out_shape=jax.ShapeDtypeStruct((N_PAD, 16), jnp.float32),
    )(z2, y2, degp, b2)


def kernel(x, edge_index, W1, b1, W2, b2):
    pad = jnp.full((NW, SLOTS - EPT), JUNK, dtype=jnp.int32)
    src_f = jnp.concatenate([edge_index[0].reshape(NW, EPT), pad], axis=1)
    dst_f = jnp.concatenate([edge_index[1].reshape(NW, EPT), pad], axis=1)
    pad64 = jnp.full((NS, _SL64 - NC * EPT), JUNK, dtype=jnp.int32)
    src_c = jnp.concatenate([edge_index[0].reshape(NS, NC * EPT), pad64], 1)
    dst_c = jnp.concatenate([edge_index[1].reshape(NS, NC * EPT), pad64], 1)
    x_pad = jnp.pad(x, ((0, N_PAD - N_NODES), (0, 0)))
    zeros32 = jnp.zeros((_CH64, 32), jnp.float32)
    zeros16 = jnp.zeros((_CH16, 16), jnp.float32)
    ones16 = jnp.ones((CHUNK, 16), jnp.float32)
    b1r = b1.reshape(1, 64)
    b2r = b2.reshape(1, 16)

    degp = _deg_kernel(dst_f.reshape(NW, NCH, CHUNK),
                       jnp.zeros((CHUNK, 16), jnp.float32),
                       ones16).reshape(NC, N_PAD, 16)
    y1 = _tc_y1(x_pad, W1, degp)
    z1 = _scatter64(src_c.reshape(NS, _SL64 // _CH64, _CH64),
                    dst_c.reshape(NS, _SL64 // _CH64, _CH64),
                    y1.reshape(NC * N_PAD, 32),
                    zeros32).reshape(NC, N_PAD, 32)
    y2 = _tc_y2(z1, y1, degp, W2, b1r)
    z2 = _scatter16(src_f.reshape(NW, _SL16 // _CH16, _CH16),
                    dst_f.reshape(NW, _SL16 // _CH16, _CH16),
                    y2, zeros16).reshape(NC, N_PAD, 16)
    out = _tc_out(z2, y2, degp, b2r)
    return out[:N_NODES]


# speedup vs baseline: 1.0308x; 1.0308x over previous
import functools

import jax
import jax.numpy as jnp
from jax import lax
from jax.experimental import pallas as pl
from jax.experimental.pallas import tpu as pltpu
from jax.experimental.pallas import tpu_sc as plsc

N_NODES = 10000
N_EDGES = 320000
NC, NS = 2, 16
NW = NC * NS
N_PAD = 10240
JUNK = N_NODES
SLOTS = 10240
CHUNK = 1024
NCH = SLOTS // CHUNK
EPT = N_EDGES // NW
ROWS_T = N_PAD // NS

_mesh = plsc.VectorSubcoreMesh(
    core_axis_name="c", subcore_axis_name="s", num_cores=NC, num_subcores=NS)


def _make_deg_kernel():
    @functools.partial(
        pl.kernel,
        out_type=jax.ShapeDtypeStruct((NC * N_PAD, 16), jnp.float32),
        mesh=_mesh,
        scratch_types=[
            pltpu.VMEM((NCH, CHUNK), jnp.int32),
            pltpu.VMEM((CHUNK, 16), jnp.float32),
            pltpu.VMEM_SHARED((N_PAD, 16), jnp.float32),
        ],
        compiler_params=pltpu.CompilerParams(use_tc_tiling_on_sc=False),
    )
    def k(dst_hbm, zeros_hbm, ones_hbm, z_hbm, dst_v, rows_v, z_sh):
        cid = lax.axis_index("c")
        sid = lax.axis_index("s")
        g = cid * NS + sid
        pltpu.sync_copy(dst_hbm.at[g], dst_v)
        pltpu.sync_copy(zeros_hbm, rows_v)
        base = sid * ROWS_T
        for kk in range(ROWS_T // 128):
            pltpu.sync_copy(rows_v.at[pl.ds(0, 128)],
                            z_sh.at[pl.ds(base + kk * 128, 128)])
        pltpu.sync_copy(ones_hbm, rows_v)
        plsc.subcore_barrier()

        def chunk_body(j, carry):
            pltpu.sync_copy(rows_v, z_sh.at[dst_v.at[j]], add=True)
            return carry

        lax.fori_loop(0, NCH, chunk_body, 0)
        plsc.subcore_barrier()
        pltpu.sync_copy(z_sh.at[pl.ds(base, ROWS_T)],
                        z_hbm.at[pl.ds(cid * N_PAD + base, ROWS_T)])

    return k


def _make_scatter_kernel(d, chunk, slots, per_core_edges):
    nch = slots // chunk

    @functools.partial(
        pl.kernel,
        out_type=jax.ShapeDtypeStruct((NC * N_PAD, d), jnp.float32),
        mesh=_mesh,
        scratch_types=[
            pltpu.VMEM((nch, chunk), jnp.int32),
            pltpu.VMEM((nch, chunk), jnp.int32),
            pltpu.VMEM((2, chunk, d), jnp.float32),
            pltpu.VMEM_SHARED((N_PAD, d), jnp.float32),
            pltpu.VMEM_SHARED((N_PAD, d), jnp.float32),
            pltpu.SemaphoreType.DMA,
        ],
        compiler_params=pltpu.CompilerParams(use_tc_tiling_on_sc=False),
    )
    def k(src_hbm, dst_hbm, y_hbm, zeros_hbm, z_hbm,
          src_v, dst_v, rows_v, z_sh, y_sh, sem):
        cid = lax.axis_index("c")
        sid = lax.axis_index("s")
        g = sid if per_core_edges else cid * NS + sid
        pltpu.sync_copy(src_hbm.at[g], src_v)
        pltpu.sync_copy(dst_hbm.at[g], dst_v)
        base = sid * ROWS_T
        yoff = cid * N_PAD + base if per_core_edges else base
        pltpu.sync_copy(y_hbm.at[pl.ds(yoff, ROWS_T)],
                        y_sh.at[pl.ds(base, ROWS_T)])
        pltpu.sync_copy(zeros_hbm, rows_v.at[0])
        for kk in range(ROWS_T // 128):
            pltpu.sync_copy(rows_v.at[0].at[pl.ds(0, 128)],
                            z_sh.at[pl.ds(base + kk * 128, 128)])
        plsc.subcore_barrier()

        pltpu.async_copy(y_sh.at[src_v.at[0]], rows_v.at[0], sem)
        for j in range(nch):
            b = j % 2
            pltpu.make_async_copy(
                y_sh.at[src_v.at[j]], rows_v.at[b], sem).wait()
            if j + 1 < nch:
                pltpu.async_copy(
                    y_sh.at[src_v.at[j + 1]], rows_v.at[1 - b], sem)
            pltpu.sync_copy(rows_v.at[b], z_sh.at[dst_v.at[j]], add=True)
        plsc.subcore_barrier()
        pltpu.sync_copy(z_sh.at[pl.ds(base, ROWS_T)],
                        z_hbm.at[pl.ds(cid * N_PAD + base, ROWS_T)])

    return k


_CH64 = 512
_SL64 = 20480
_CH16 = 1024
_SL16 = SLOTS

_deg_kernel = _make_deg_kernel()
_scatter64 = _make_scatter_kernel(32, _CH64, _SL64, per_core_edges=True)
_scatter16 = _make_scatter_kernel(16, _CH16, _SL16, per_core_edges=False)

def _dis_of(dp_ref):
    deg = dp_ref[0, :, 0:1] + dp_ref[1, :, 0:1] + 1.0
    return lax.rsqrt(deg)


def _y1_body(x_ref, w_ref, dp_ref, y_ref):
    dis = _dis_of(dp_ref)[0:N_NODES]
    y = jnp.dot(x_ref[...] * dis, w_ref[...],
                preferred_element_type=jnp.float32)
    y_ref[0, pl.ds(0, N_NODES)] = y[:, 0:32]
    y_ref[1, pl.ds(0, N_NODES)] = y[:, 32:64]


def _tc_y1(x_pad, W1, degp):
    return pl.pallas_call(
        _y1_body,
        out_shape=jax.ShapeDtypeStruct((2, N_PAD, 32), jnp.float32),
    )(x_pad, W1, degp)


def _y2_body(z_ref, y1_ref, dp_ref, w_ref, b_ref, y2_ref):
    dis = _dis_of(dp_ref)
    zc = jnp.concatenate([z_ref[0], z_ref[1]], axis=-1)
    yc = jnp.concatenate([y1_ref[0], y1_ref[1]], axis=-1)
    s = dis * (zc + yc) + b_ref[...]
    h = jnp.maximum(s, 0.0) * dis
    y2_ref[...] = jnp.dot(h, w_ref[...], preferred_element_type=jnp.float32)


def _tc_y2(z1, y1, degp, W2, b1):
    return pl.pallas_call(
        _y2_body,
        out_shape=jax.ShapeDtypeStruct((N_PAD, 16), jnp.float32),
    )(z1, y1, degp, W2, b1)


def _out_body(z_ref, y2_ref, dp_ref, b_ref, o_ref):
    dis = _dis_of(dp_ref)
    s = dis * (z_ref[0] + z_ref[1] + y2_ref[...]) + b_ref[...]
    m = jnp.max(s, axis=1, keepdims=True)
    e = jnp.exp(s - m)
    o_ref[...] = s - (m + jnp.log(jnp.sum(e, axis=1, keepdims=True)))


def _tc_out(z2, y2, degp, b2):
    return pl.pallas_call(
        _out_body,
        out_shape=jax.ShapeDtypeStruct((N_PAD, 16), jnp.float32),
    )(z2, y2, degp, b2)


def kernel(x, edge_index, W1, b1, W2, b2):
    pad = jnp.full((NW, SLOTS - EPT), JUNK, dtype=jnp.int32)
    src_f = jnp.concatenate([edge_index[0].reshape(NW, EPT), pad], axis=1)
    dst_f = jnp.concatenate([edge_index[1].reshape(NW, EPT), pad], axis=1)
    pad64 = jnp.full((NS, _SL64 - NC * EPT), JUNK, dtype=jnp.int32)
    src_c = jnp.concatenate([edge_index[0].reshape(NS, NC * EPT), pad64], 1)
    dst_c = jnp.concatenate([edge_index[1].reshape(NS, NC * EPT), pad64], 1)
    zeros32 = jnp.zeros((_CH64, 32), jnp.float32)
    zeros16 = jnp.zeros((_CH16, 16), jnp.float32)
    ones16 = jnp.ones((CHUNK, 16), jnp.float32)
    b1r = b1.reshape(1, 64)
    b2r = b2.reshape(1, 16)

    degp = _deg_kernel(dst_f.reshape(NW, NCH, CHUNK),
                       jnp.zeros((CHUNK, 16), jnp.float32),
                       ones16).reshape(NC, N_PAD, 16)
    y1 = _tc_y1(x, W1, degp)
    z1 = _scatter64(src_c.reshape(NS, _SL64 // _CH64, _CH64),
                    dst_c.reshape(NS, _SL64 // _CH64, _CH64),
                    y1.reshape(NC * N_PAD, 32),
                    zeros32).reshape(NC, N_PAD, 32)
    y2 = _tc_y2(z1, y1, degp, W2, b1r)
    z2 = _scatter16(src_f.reshape(NW, _SL16 // _CH16, _CH16),
                    dst_f.reshape(NW, _SL16 // _CH16, _CH16),
                    y2, zeros16).reshape(NC, N_PAD, 16)
    out = _tc_out(z2, y2, degp, b2r)
    return out[:N_NODES]
